# Initial kernel scaffold; baseline (speedup 1.0000x reference)
import jax
import jax.numpy as jnp
from jax.experimental import pallas as pl


def kernel(ixs, targets, table):
    raise NotImplementedError("write your pallas kernel here")


# speedup vs baseline: 1.4891x; 1.4891x over previous
import functools

import jax
import jax.numpy as jnp
from jax import lax
from jax.experimental import pallas as pl
from jax.experimental.pallas import tpu as pltpu
from jax.experimental.pallas import tpu_sc as plsc

V = 1000
N = 1024 * 50

_info = plsc.get_sparse_core_info()
NC = _info.num_cores
NS = _info.num_subcores
L = _info.num_lanes
NW = NC * NS
RPW = N // NW
C = 64
NCHUNK = RPW // C


def _logp_body(table_ref, logp_ref):
    x = table_ref[...]
    m = jnp.max(x, axis=1, keepdims=True)
    s = jnp.sum(jnp.exp(x - m), axis=1, keepdims=True)
    logp_ref[...] = x - (m + jnp.log(s))


def _table_logp(table):
    return pl.pallas_call(
        _logp_body,
        out_shape=jax.ShapeDtypeStruct((V, V), jnp.float32),
    )(table)


_mesh = plsc.VectorSubcoreMesh(core_axis_name="c", subcore_axis_name="s")


@functools.partial(
    pl.kernel,
    out_type=(
        jax.ShapeDtypeStruct((N, V), jnp.float32),
        jax.ShapeDtypeStruct((NW, L), jnp.float32),
    ),
    mesh=_mesh,
    compiler_params=pltpu.CompilerParams(use_tc_tiling_on_sc=False),
    scratch_types=[
        pltpu.VMEM((C,), jnp.int32),
        pltpu.VMEM((C,), jnp.int32),
        pltpu.VMEM((C,), jnp.int32),
        pltpu.VMEM((C, V), jnp.float32),
        pltpu.VMEM((C,), jnp.float32),
        pltpu.VMEM((L,), jnp.float32),
        pltpu.SemaphoreType.DMA,
    ],
)
def _sc_gather_nll(ixs_hbm, tgt_hbm, table_hbm, logp_hbm, out_hbm, part_hbm,
                   idx_v, tgt_v, lin_v, rows_v, val_v, acc_v, sem):
    wid = lax.axis_index("s") * NC + lax.axis_index("c")
    acc_v[...] = jnp.zeros((L,), jnp.float32)

    def chunk(c, carry):
        base = wid * RPW + c * C
        pltpu.sync_copy(ixs_hbm.at[pl.ds(base, C)], idx_v)
        pltpu.sync_copy(tgt_hbm.at[pl.ds(base, C)], tgt_v)
        for j in range(C // L):
            il = idx_v[pl.ds(j * L, L)]
            tl = tgt_v[pl.ds(j * L, L)]
            lin_v[pl.ds(j * L, L)] = il * V + tl
        rows_cp = pltpu.async_copy(table_hbm.at[idx_v], rows_v, sem)
        vals_cp = pltpu.async_copy(logp_hbm.at[lin_v], val_v, sem)
        rows_cp.wait()
        vals_cp.wait()
        pltpu.sync_copy(rows_v, out_hbm.at[pl.ds(base, C)])
        for j in range(C // L):
            acc_v[...] = acc_v[...] + val_v[pl.ds(j * L, L)]
        return carry

    lax.fori_loop(0, NCHUNK, chunk, 0)
    pltpu.sync_copy(acc_v, part_hbm.at[wid])


def kernel(ixs, targets, table):
    b, t = ixs.shape
    logp = _table_logp(table).reshape(-1)
    logits_f, part = _sc_gather_nll(
        ixs.reshape(-1), targets.reshape(-1), table, logp)
    loss = -jnp.sum(part) / (b * t)
    return (logits_f.reshape(b, t, V), loss)


# speedup vs baseline: 1.5320x; 1.0288x over previous
import functools

import jax
import jax.numpy as jnp
from jax import lax
from jax.experimental import pallas as pl
from jax.experimental.pallas import tpu as pltpu
from jax.experimental.pallas import tpu_sc as plsc

V = 1000
N = 1024 * 50

_info = plsc.get_sparse_core_info()
NC = _info.num_cores
NS = _info.num_subcores
L = _info.num_lanes
NW = NC * NS
RPW = N // NW
C = 32
NCHUNK = RPW // C
PAIRS = NCHUNK // 2


def _logp_body(table_ref, logp_ref):
    x = table_ref[...]
    m = jnp.max(x, axis=1, keepdims=True)
    s = jnp.sum(jnp.exp(x - m), axis=1, keepdims=True)
    logp_ref[...] = x - (m + jnp.log(s))


def _table_logp(table):
    return pl.pallas_call(
        _logp_body,
        out_shape=jax.ShapeDtypeStruct((V, V), jnp.float32),
    )(table)


_mesh = plsc.VectorSubcoreMesh(core_axis_name="c", subcore_axis_name="s")


@functools.partial(
    pl.kernel,
    out_type=(
        jax.ShapeDtypeStruct((N, V), jnp.float32),
        jax.ShapeDtypeStruct((NW, L), jnp.float32),
    ),
    mesh=_mesh,
    compiler_params=pltpu.CompilerParams(use_tc_tiling_on_sc=False),
    scratch_types=[
        pltpu.VMEM((RPW,), jnp.int32),
        pltpu.VMEM((RPW,), jnp.int32),
        pltpu.VMEM((2, C, V), jnp.float32),
        pltpu.VMEM((2, C), jnp.float32),
        pltpu.VMEM((L,), jnp.float32),
        pltpu.SemaphoreType.DMA,
        pltpu.SemaphoreType.DMA,
        pltpu.SemaphoreType.DMA,
        pltpu.SemaphoreType.DMA,
    ],
)
def _sc_gather_nll(ixs_hbm, tgt_hbm, table_hbm, logp_hbm, out_hbm, part_hbm,
                   idx_all, lin_all, rows_v, val_v, acc_v,
                   sem_g0, sem_g1, sem_s0, sem_s1):
    wid = lax.axis_index("s") * NC + lax.axis_index("c")
    wbase = wid * RPW
    sems_g = (sem_g0, sem_g1)
    sems_s = (sem_s0, sem_s1)

    pltpu.sync_copy(ixs_hbm.at[pl.ds(wbase, RPW)], idx_all)
    pltpu.sync_copy(tgt_hbm.at[pl.ds(wbase, RPW)], lin_all)

    def linbody(j, carry):
        il = idx_all[pl.ds(j * L, L)]
        tl = lin_all[pl.ds(j * L, L)]
        lin_all[pl.ds(j * L, L)] = il * V + tl
        return carry

    lax.fori_loop(0, RPW // L, linbody, 0)
    acc_v[...] = jnp.zeros((L,), jnp.float32)

    def start_gather(c, b):
        off = c * C
        pltpu.async_copy(table_hbm.at[idx_all.at[pl.ds(off, C)]],
                         rows_v.at[b], sems_g[b])
        pltpu.async_copy(logp_hbm.at[lin_all.at[pl.ds(off, C)]],
                         val_v.at[b], sems_g[b])

    def wait_gather(b):
        pltpu.make_async_copy(table_hbm.at[idx_all.at[pl.ds(0, C)]],
                              rows_v.at[b], sems_g[b]).wait()
        pltpu.make_async_copy(logp_hbm.at[lin_all.at[pl.ds(0, C)]],
                              val_v.at[b], sems_g[b]).wait()

    def wait_scatter(b):
        pltpu.make_async_copy(rows_v.at[b], out_hbm.at[pl.ds(0, C)],
                              sems_s[b]).wait()

    start_gather(0, 0)

    def pair(g, carry):
        for b in (0, 1):
            c = 2 * g + b
            nb = 1 - b
            wait_gather(b)
            pltpu.async_copy(rows_v.at[b],
                             out_hbm.at[pl.ds(wbase + c * C, C)], sems_s[b])
            for j in range(C // L):
                acc_v[...] = acc_v[...] + val_v.at[b][pl.ds(j * L, L)]
            if b == 0:
                @pl.when(g > 0)
                def _():
                    wait_scatter(nb)
                start_gather(c + 1, nb)
            else:
                wait_scatter(nb)
                @pl.when(g < PAIRS - 1)
                def _():
                    start_gather(c + 1, nb)
        return carry

    lax.fori_loop(0, PAIRS, pair, 0)
    wait_scatter(1)
    pltpu.sync_copy(acc_v, part_hbm.at[wid])


def kernel(ixs, targets, table):
    b, t = ixs.shape
    logp = _table_logp(table).reshape(-1)
    logits_f, part = _sc_gather_nll(
        ixs.reshape(-1), targets.reshape(-1), table, logp)
    loss = -jnp.sum(part) / (b * t)
    return (logits_f.reshape(b, t, V), loss)
